# baseline (speedup 1.0000x reference)
import functools

import jax
import jax.numpy as jnp
from jax import lax
from jax.experimental import pallas as pl
from jax.experimental.pallas import tpu as pltpu
from jax.experimental.pallas import tpu_sc as plsc

_NC = 2
_NS = 16
_NW = _NC * _NS


def _build(n, d, dtype):
    per_w = n // _NW
    ch = 128
    n_ch = per_w // ch

    mesh = plsc.VectorSubcoreMesh(core_axis_name="c", subcore_axis_name="s")

    @functools.partial(
        pl.kernel,
        out_type=jax.ShapeDtypeStruct((_NW, n_ch, ch, d), dtype),
        mesh=mesh,
        scratch_types=[
            pltpu.VMEM((n_ch, ch), jnp.int32),
            pltpu.VMEM((ch, d), dtype),
            pltpu.SemaphoreType.DMA,
        ],
        compiler_params=pltpu.CompilerParams(use_tc_tiling_on_sc=False),
    )
    def emb(idx_hbm, table_hbm, out_hbm, idx_v, rows_v, gsem):
        wid = lax.axis_index("s") * _NC + lax.axis_index("c")
        pltpu.sync_copy(idx_hbm.at[wid], idx_v)

        @pl.loop(0, n_ch)
        def _chunk(j):
            pltpu.async_copy(table_hbm.at[idx_v.at[j]], rows_v, gsem).wait()
            pltpu.sync_copy(rows_v, out_hbm.at[wid, j])

    return emb


def kernel(x, table):
    b, s = x.shape
    v, d = table.shape
    n = b * s
    per_w = n // _NW
    ch = 128
    n_ch = per_w // ch
    idx = x.reshape(_NW, n_ch, ch).astype(jnp.int32)
    out = _build(n, d, table.dtype)(idx, table)
    return out.reshape(b, s, d)


# speedup vs baseline: 1.1145x; 1.1145x over previous
import functools

import jax
import jax.numpy as jnp
from jax import lax
from jax.experimental import pallas as pl
from jax.experimental.pallas import tpu as pltpu
from jax.experimental.pallas import tpu_sc as plsc

_NC = 2
_NS = 16
_NW = _NC * _NS


def _build(n, d, dtype):
    per_w = n // _NW
    ch = 128
    n_ch = per_w // ch
    nbuf = 8

    mesh = plsc.VectorSubcoreMesh(core_axis_name="c", subcore_axis_name="s")

    @functools.partial(
        pl.kernel,
        out_type=jax.ShapeDtypeStruct((_NW, n_ch, ch, d), dtype),
        mesh=mesh,
        scratch_types=[
            pltpu.VMEM((n_ch, ch), jnp.int32),
            pltpu.VMEM((nbuf, ch, d), dtype),
            [pltpu.SemaphoreType.DMA] * nbuf,
        ],
        compiler_params=pltpu.CompilerParams(use_tc_tiling_on_sc=False),
    )
    def emb(idx_hbm, table_hbm, out_hbm, idx_v, rows_v, gsems):
        wid = lax.axis_index("s") * _NC + lax.axis_index("c")
        pltpu.sync_copy(idx_hbm.at[wid], idx_v)

        for b in range(nbuf):
            pltpu.async_copy(table_hbm.at[idx_v.at[b]], rows_v.at[b], gsems[b])

        @pl.loop(0, n_ch, step=nbuf)
        def _group(g):
            for b in range(nbuf):
                j = g + b
                pltpu.make_async_copy(
                    table_hbm.at[idx_v.at[j]], rows_v.at[b], gsems[b]
                ).wait()
                pltpu.sync_copy(rows_v.at[b], out_hbm.at[wid, j])

                @pl.when(j + nbuf < n_ch)
                def _prefetch():
                    pltpu.async_copy(
                        table_hbm.at[idx_v.at[j + nbuf]], rows_v.at[b], gsems[b]
                    )

    return emb


def kernel(x, table):
    b, s = x.shape
    v, d = table.shape
    n = b * s
    per_w = n // _NW
    ch = 128
    n_ch = per_w // ch
    idx = x.reshape(_NW, n_ch, ch).astype(jnp.int32)
    out = _build(n, d, table.dtype)(idx, table)
    return out.reshape(b, s, d)
